# initial kernel scaffold (unmeasured)
import jax
import jax.numpy as jnp
from jax import lax
from jax.experimental import pallas as pl
from jax.experimental.pallas import tpu as pltpu

N_DEV = 16


def kernel(x, Wq, K_ext, V_ext, Wo):
    B, Sq, D = x.shape
    _, Skv, Hq, Dh = K_ext.shape
    Dm = Hq * Dh
    Do = Wo.shape[1]
    QB = Sq // 64
    ROWS = Sq // N_DEV

    Kf = K_ext.reshape(B, Skv, Dm)
    Vf = V_ext.reshape(B, Skv, Dm)

    def body(x_ref, wq_ref, k_ref, v_ref, wo_ref, out_ref,
             accC, accE, recvC, recvE,
             s1C, s1E, r1C, r1E, s2, r2):
        me = lax.axis_index("i")

        for b in range(B):
            Qm = jnp.dot(x_ref[b], wq_ref[...],
                         preferred_element_type=jnp.float32)
            for q in range(QB):
                rows = pl.ds(q * 64, 64)
                ses = []
                for h in range(Hq):
                    cols = pl.ds(h * Dh, Dh)
                    Qh = Qm[q * 64:(q + 1) * 64, h * Dh:(h + 1) * Dh]
                    Kh = k_ref[b, rows, cols]
                    Vh = v_ref[b, rows, cols]
                    s = lax.dot_general(
                        Qh, Kh, (((1,), (1,)), ((), ())),
                        preferred_element_type=jnp.float32) * 0.125
                    w = jnp.exp(s)
                    ses.append(jnp.sum(w, axis=1))
                    ctx = lax.dot_general(
                        w, Vh, (((1,), (0,)), ((), ())),
                        preferred_element_type=jnp.float32)
                    accC[b, rows, cols] = ctx
                accE[b, rows, :] = jnp.stack(ses, axis=1)

        recvC[me] = accC[:, pl.ds(me * ROWS, ROWS), :]
        recvE[me] = accE[:, pl.ds(me * ROWS, ROWS), :]

        p1 = []
        for off in range(1, N_DEV):
            peer = lax.rem(me + off, N_DEV)
            rC = pltpu.make_async_remote_copy(
                src_ref=accC.at[:, pl.ds(peer * ROWS, ROWS), :],
                dst_ref=recvC.at[me],
                send_sem=s1C.at[off], recv_sem=r1C.at[me],
                device_id=(peer,), device_id_type=pl.DeviceIdType.MESH)
            rE = pltpu.make_async_remote_copy(
                src_ref=accE.at[:, pl.ds(peer * ROWS, ROWS), :],
                dst_ref=recvE.at[me],
                send_sem=s1E.at[off], recv_sem=r1E.at[me],
                device_id=(peer,), device_id_type=pl.DeviceIdType.MESH)
            rC.start()
            rE.start()
            p1.append((rC, rE))

        for off in range(1, N_DEV):
            src = lax.rem(me + off, N_DEV)
            pltpu.make_async_remote_copy(
                src_ref=accC.at[:, pl.ds(0, ROWS), :],
                dst_ref=recvC.at[src],
                send_sem=s1C.at[off], recv_sem=r1C.at[src],
                device_id=(src,), device_id_type=pl.DeviceIdType.MESH,
            ).wait_recv()
            pltpu.make_async_remote_copy(
                src_ref=accE.at[:, pl.ds(0, ROWS), :],
                dst_ref=recvE.at[src],
                send_sem=s1E.at[off], recv_sem=r1E.at[src],
                device_id=(src,), device_id_type=pl.DeviceIdType.MESH,
            ).wait_recv()

        RC = jnp.sum(recvC[...], axis=0)
        RE = jnp.sum(recvE[...], axis=0)
        Nrm = RC.reshape(B, ROWS, Hq, Dh) / RE[..., None]
        y = Nrm.reshape(B, ROWS, Dm)
        for b in range(B):
            out_ref[b, pl.ds(me * ROWS, ROWS), :] = jnp.dot(
                y[b], wo_ref[...], preferred_element_type=jnp.float32)

        p2 = []
        for off in range(1, N_DEV):
            peer = lax.rem(me + off, N_DEV)
            r = pltpu.make_async_remote_copy(
                src_ref=out_ref.at[:, pl.ds(me * ROWS, ROWS), :],
                dst_ref=out_ref.at[:, pl.ds(me * ROWS, ROWS), :],
                send_sem=s2.at[off], recv_sem=r2.at[me],
                device_id=(peer,), device_id_type=pl.DeviceIdType.MESH)
            r.start()
            p2.append(r)

        for off in range(1, N_DEV):
            src = lax.rem(me + off, N_DEV)
            pltpu.make_async_remote_copy(
                src_ref=out_ref.at[:, pl.ds(src * ROWS, ROWS), :],
                dst_ref=out_ref.at[:, pl.ds(src * ROWS, ROWS), :],
                send_sem=s2.at[off], recv_sem=r2.at[src],
                device_id=(src,), device_id_type=pl.DeviceIdType.MESH,
            ).wait_recv()

        for rC, rE in p1:
            rC.wait_send()
            rE.wait_send()
        for r in p2:
            r.wait_send()

    out_shape = jax.ShapeDtypeStruct((B, Sq, Do), jnp.float32)
    return pl.pallas_call(
        body,
        out_shape=out_shape,
        in_specs=[pl.BlockSpec(memory_space=pltpu.VMEM)] * 5,
        out_specs=pl.BlockSpec(memory_space=pltpu.VMEM),
        scratch_shapes=[
            pltpu.VMEM((B, Sq, Dm), jnp.float32),
            pltpu.VMEM((B, Sq, Hq), jnp.float32),
            pltpu.VMEM((N_DEV, B, ROWS, Dm), jnp.float32),
            pltpu.VMEM((N_DEV, B, ROWS, Hq), jnp.float32),
            pltpu.SemaphoreType.DMA((N_DEV,)),
            pltpu.SemaphoreType.DMA((N_DEV,)),
            pltpu.SemaphoreType.DMA((N_DEV,)),
            pltpu.SemaphoreType.DMA((N_DEV,)),
            pltpu.SemaphoreType.DMA((N_DEV,)),
            pltpu.SemaphoreType.DMA((N_DEV,)),
        ],
        compiler_params=pltpu.CompilerParams(collective_id=0),
    )(x, Wq, Kf, Vf, Wo)


# baseline (device time: 38307 ns/iter reference)
import jax
import jax.numpy as jnp
from jax import lax
from jax.experimental import pallas as pl
from jax.experimental.pallas import tpu as pltpu

N_DEV = 16


def kernel(x, Wq, K_ext, V_ext, Wo):
    B, Sq, D = x.shape
    _, Skv, Hq, Dh = K_ext.shape
    Dm = Hq * Dh
    Do = Wo.shape[1]
    QB = Sq // 64
    ROWS = Sq // N_DEV

    Kf = K_ext.reshape(B, Skv, Dm)
    Vf = V_ext.reshape(B, Skv, Dm)

    def body(x_ref, wq_ref, k_ref, v_ref, wo_ref, out_ref,
             accC, accE, recvC, recvE,
             s1C, s1E, r1C, r1E, s2, r2):
        me = lax.axis_index("i")

        for b in range(B):
            Qm = jnp.dot(x_ref[b], wq_ref[...],
                         preferred_element_type=jnp.float32)
            for q in range(QB):
                rows = pl.ds(q * 64, 64)
                ses = []
                for h in range(Hq):
                    cols = pl.ds(h * Dh, Dh)
                    Qh = Qm[q * 64:(q + 1) * 64, h * Dh:(h + 1) * Dh]
                    Kh = k_ref[b, rows, cols]
                    Vh = v_ref[b, rows, cols]
                    s = lax.dot_general(
                        Qh, Kh, (((1,), (1,)), ((), ())),
                        preferred_element_type=jnp.float32) * 0.125
                    w = jnp.exp(s)
                    ses.append(jnp.sum(w, axis=1))
                    ctx = lax.dot_general(
                        w, Vh, (((1,), (0,)), ((), ())),
                        preferred_element_type=jnp.float32)
                    accC[b, rows, cols] = ctx
                accE[b, rows, :] = jnp.stack(ses, axis=1)

        recvC[me] = accC[:, pl.ds(me * ROWS, ROWS), :]
        recvE[me] = accE[:, pl.ds(me * ROWS, ROWS), :]

        p1 = []
        for off in range(1, N_DEV):
            peer = lax.rem(me + off, N_DEV)
            rC = pltpu.make_async_remote_copy(
                src_ref=accC.at[:, pl.ds(peer * ROWS, ROWS), :],
                dst_ref=recvC.at[me],
                send_sem=s1C.at[off], recv_sem=r1C.at[me],
                device_id=(peer,), device_id_type=pl.DeviceIdType.MESH)
            rE = pltpu.make_async_remote_copy(
                src_ref=accE.at[:, pl.ds(peer * ROWS, ROWS), :],
                dst_ref=recvE.at[me],
                send_sem=s1E.at[off], recv_sem=r1E.at[me],
                device_id=(peer,), device_id_type=pl.DeviceIdType.MESH)
            rC.start()
            rE.start()
            p1.append((rC, rE))

        for off in range(1, N_DEV):
            src = lax.rem(me + off, N_DEV)
            pltpu.make_async_remote_copy(
                src_ref=accC.at[:, pl.ds(0, ROWS), :],
                dst_ref=recvC.at[src],
                send_sem=s1C.at[off], recv_sem=r1C.at[src],
                device_id=(src,), device_id_type=pl.DeviceIdType.MESH,
            ).wait_recv()
            pltpu.make_async_remote_copy(
                src_ref=accE.at[:, pl.ds(0, ROWS), :],
                dst_ref=recvE.at[src],
                send_sem=s1E.at[off], recv_sem=r1E.at[src],
                device_id=(src,), device_id_type=pl.DeviceIdType.MESH,
            ).wait_recv()

        RC = jnp.sum(recvC[...], axis=0)
        RE = jnp.sum(recvE[...], axis=0)
        Nrm = RC.reshape(B, ROWS, Hq, Dh) / RE[..., None]
        y = Nrm.reshape(B, ROWS, Dm)
        for b in range(B):
            out_ref[b, pl.ds(me * ROWS, ROWS), :] = jnp.dot(
                y[b], wo_ref[...], preferred_element_type=jnp.float32)

        p2 = []
        for off in range(1, N_DEV):
            peer = lax.rem(me + off, N_DEV)
            r = pltpu.make_async_remote_copy(
                src_ref=out_ref.at[:, pl.ds(me * ROWS, ROWS), :],
                dst_ref=out_ref.at[:, pl.ds(me * ROWS, ROWS), :],
                send_sem=s2.at[off], recv_sem=r2.at[me],
                device_id=(peer,), device_id_type=pl.DeviceIdType.MESH)
            r.start()
            p2.append(r)

        for off in range(1, N_DEV):
            src = lax.rem(me + off, N_DEV)
            pltpu.make_async_remote_copy(
                src_ref=out_ref.at[:, pl.ds(src * ROWS, ROWS), :],
                dst_ref=out_ref.at[:, pl.ds(src * ROWS, ROWS), :],
                send_sem=s2.at[off], recv_sem=r2.at[src],
                device_id=(src,), device_id_type=pl.DeviceIdType.MESH,
            ).wait_recv()

        for rC, rE in p1:
            rC.wait_send()
            rE.wait_send()
        for r in p2:
            r.wait_send()

    out_shape = jax.ShapeDtypeStruct((B, Sq, Do), jnp.float32)
    return pl.pallas_call(
        body,
        out_shape=out_shape,
        in_specs=[pl.BlockSpec(memory_space=pltpu.VMEM)] * 5,
        out_specs=pl.BlockSpec(memory_space=pltpu.VMEM),
        scratch_shapes=[
            pltpu.VMEM((B, Sq, Dm), jnp.float32),
            pltpu.VMEM((B, Sq, Hq), jnp.float32),
            pltpu.VMEM((N_DEV, B, ROWS, Dm), jnp.float32),
            pltpu.VMEM((N_DEV, B, ROWS, Hq), jnp.float32),
            pltpu.SemaphoreType.DMA((N_DEV,)),
            pltpu.SemaphoreType.DMA((N_DEV,)),
            pltpu.SemaphoreType.DMA((N_DEV,)),
            pltpu.SemaphoreType.DMA((N_DEV,)),
            pltpu.SemaphoreType.DMA((N_DEV,)),
            pltpu.SemaphoreType.DMA((N_DEV,)),
        ],
    )(x, Wq, Kf, Vf, Wo)


# device time: 9422 ns/iter; 4.0657x vs baseline; 4.0657x over previous
import jax
import jax.numpy as jnp
from jax import lax
from jax.experimental import pallas as pl
from jax.experimental.pallas import tpu as pltpu

N_DEV = 16


def kernel(x, Wq, K_ext, V_ext, Wo):
    B, Sq, D = x.shape
    _, Skv, Hq, Dh = K_ext.shape
    Dm = Hq * Dh
    Do = Wo.shape[1]
    QB = Sq // 64
    ROWS = Sq // N_DEV

    Kf = K_ext.reshape(B, Skv, Dm)
    Vf = V_ext.reshape(B, Skv, Dm)

    def body(x_ref, wq_ref, k_ref, v_ref, wo_ref, out_ref,
             accC, accE, recvC, recvE, ybuf,
             s1C, s1E, r1C, r1E, s2, r2):
        me = lax.axis_index("i")

        for b in range(B):
            Qm = jnp.dot(x_ref[b], wq_ref[...],
                         preferred_element_type=jnp.float32)
            for q in range(QB):
                rows = pl.ds(q * 64, 64)
                ses = []
                for h in range(Hq):
                    cols = pl.ds(h * Dh, Dh)
                    Qh = Qm[q * 64:(q + 1) * 64, h * Dh:(h + 1) * Dh]
                    Kh = k_ref[b, rows, cols]
                    Vh = v_ref[b, rows, cols]
                    s = lax.dot_general(
                        Qh, Kh, (((1,), (1,)), ((), ())),
                        preferred_element_type=jnp.float32) * 0.125
                    w = jnp.exp(s)
                    ses.append(jnp.sum(w, axis=1))
                    ctx = lax.dot_general(
                        w, Vh, (((1,), (0,)), ((), ())),
                        preferred_element_type=jnp.float32)
                    accC[b, rows, cols] = ctx
                accE[b, rows, :] = jnp.stack(ses, axis=1)

        recvC[me] = accC[:, pl.ds(me * ROWS, ROWS), :]
        recvE[me] = accE[:, pl.ds(me * ROWS, ROWS), :]

        p1 = []
        for off in range(1, N_DEV):
            peer = lax.rem(me + off, N_DEV)
            rC = pltpu.make_async_remote_copy(
                src_ref=accC.at[:, pl.ds(peer * ROWS, ROWS), :],
                dst_ref=recvC.at[me],
                send_sem=s1C.at[off], recv_sem=r1C.at[me],
                device_id=(peer,), device_id_type=pl.DeviceIdType.MESH)
            rE = pltpu.make_async_remote_copy(
                src_ref=accE.at[:, pl.ds(peer * ROWS, ROWS), :],
                dst_ref=recvE.at[me],
                send_sem=s1E.at[off], recv_sem=r1E.at[me],
                device_id=(peer,), device_id_type=pl.DeviceIdType.MESH)
            rC.start()
            rE.start()
            p1.append((rC, rE))

        for off in range(1, N_DEV):
            src = lax.rem(me + off, N_DEV)
            pltpu.make_async_remote_copy(
                src_ref=accC.at[:, pl.ds(0, ROWS), :],
                dst_ref=recvC.at[src],
                send_sem=s1C.at[off], recv_sem=r1C.at[src],
                device_id=(src,), device_id_type=pl.DeviceIdType.MESH,
            ).wait_recv()
            pltpu.make_async_remote_copy(
                src_ref=accE.at[:, pl.ds(0, ROWS), :],
                dst_ref=recvE.at[src],
                send_sem=s1E.at[off], recv_sem=r1E.at[src],
                device_id=(src,), device_id_type=pl.DeviceIdType.MESH,
            ).wait_recv()

        RC = jnp.sum(recvC[...], axis=0)
        RE = jnp.sum(recvE[...], axis=0)
        Nrm = RC.reshape(B, ROWS, Hq, Dh) / RE[..., None]
        ybuf[:, pl.ds(me * ROWS, ROWS), :] = Nrm.reshape(B, ROWS, Dm)

        p2 = []
        for off in range(1, N_DEV):
            peer = lax.rem(me + off, N_DEV)
            r = pltpu.make_async_remote_copy(
                src_ref=ybuf.at[:, pl.ds(me * ROWS, ROWS), :],
                dst_ref=ybuf.at[:, pl.ds(me * ROWS, ROWS), :],
                send_sem=s2.at[off], recv_sem=r2.at[me],
                device_id=(peer,), device_id_type=pl.DeviceIdType.MESH)
            r.start()
            p2.append(r)

        for b in range(B):
            out_ref[b, pl.ds(me * ROWS, ROWS), :] = jnp.dot(
                ybuf[b, pl.ds(me * ROWS, ROWS), :], wo_ref[...],
                preferred_element_type=jnp.float32)

        for off in range(1, N_DEV):
            src = lax.rem(me + off, N_DEV)
            pltpu.make_async_remote_copy(
                src_ref=ybuf.at[:, pl.ds(src * ROWS, ROWS), :],
                dst_ref=ybuf.at[:, pl.ds(src * ROWS, ROWS), :],
                send_sem=s2.at[off], recv_sem=r2.at[src],
                device_id=(src,), device_id_type=pl.DeviceIdType.MESH,
            ).wait_recv()
            for b in range(B):
                out_ref[b, pl.ds(src * ROWS, ROWS), :] = jnp.dot(
                    ybuf[b, pl.ds(src * ROWS, ROWS), :], wo_ref[...],
                    preferred_element_type=jnp.float32)

        for rC, rE in p1:
            rC.wait_send()
            rE.wait_send()
        for r in p2:
            r.wait_send()

    out_shape = jax.ShapeDtypeStruct((B, Sq, Do), jnp.float32)
    return pl.pallas_call(
        body,
        out_shape=out_shape,
        in_specs=[pl.BlockSpec(memory_space=pltpu.VMEM)] * 5,
        out_specs=pl.BlockSpec(memory_space=pltpu.VMEM),
        scratch_shapes=[
            pltpu.VMEM((B, Sq, Dm), jnp.float32),
            pltpu.VMEM((B, Sq, Hq), jnp.float32),
            pltpu.VMEM((N_DEV, B, ROWS, Dm), jnp.float32),
            pltpu.VMEM((N_DEV, B, ROWS, Hq), jnp.float32),
            pltpu.VMEM((B, Sq, Dm), jnp.float32),
            pltpu.SemaphoreType.DMA((N_DEV,)),
            pltpu.SemaphoreType.DMA((N_DEV,)),
            pltpu.SemaphoreType.DMA((N_DEV,)),
            pltpu.SemaphoreType.DMA((N_DEV,)),
            pltpu.SemaphoreType.DMA((N_DEV,)),
            pltpu.SemaphoreType.DMA((N_DEV,)),
        ],
    )(x, Wq, Kf, Vf, Wo)
